# Initial kernel scaffold; baseline (speedup 1.0000x reference)
import jax
import jax.numpy as jnp
from jax.experimental import pallas as pl


def kernel(x, M_init, W_kv, W_q, B_gates):
    raise NotImplementedError("write your pallas kernel here")


# speedup vs baseline: 2.3505x; 2.3505x over previous
import jax
import jax.numpy as jnp
from jax.experimental import pallas as pl
from jax.experimental.pallas import tpu as pltpu

T, B, DIM = 2048, 16, 1024
N, K = 64, 3
P = K * 2 * N + N
BB = 8
TT = 256
NT = T // TT


def _e83_kernel(x_ref, wt_ref, bgs_ref, bgl_ref, minit_ref,
                out_ref, mfin_ref, proj_s, m_s):
    t_blk = pl.program_id(1)

    @pl.when(t_blk == 0)
    def _():
        m_s[...] = minit_ref[...]

    xb = x_ref[...].reshape(TT * BB, DIM)
    proj_s[...] = jnp.dot(
        xb, wt_ref[...], preferred_element_type=jnp.float32
    ).reshape(TT, BB, P)

    def step(t, _):
        row = proj_s[t]
        q = row[:, K * 2 * N:]
        m0_old = m_s[0]
        for k in range(K):
            kk = row[:, 2 * N * k: 2 * N * k + N]
            vv = row[:, 2 * N * k + N: 2 * N * (k + 1)]
            nrm = jnp.sqrt(jnp.sum(kk * kk, axis=-1, keepdims=True))
            kn = kk / (nrm + 1e-6)
            G = m0_old if k == K - 1 else m_s[k + 1]
            Mk = m0_old if k == 0 else m_s[k]
            knb = kn[:, None, :]
            gks = jnp.sum(G * knb, axis=-1, keepdims=True)
            rets = jnp.sum(Mk * knb, axis=-1, keepdims=True)
            knT = kn[:, :, None]
            cgs = jnp.sum(G * knT, axis=1, keepdims=True)
            rg = jax.nn.sigmoid(gks + bgs_ref[k][None])
            cg = jax.nn.sigmoid(cgs + bgl_ref[k][None])
            delta = vv[:, :, None] - rets
            m_new = rg * Mk * cg + delta * knb
            m_s[k] = m_new
            if k == 0:
                sq = jnp.sum(m_new * q[:, None, :], axis=-1, keepdims=True)
                o = sq * sq * jax.nn.sigmoid(sq)
                out_ref[t] = o[:, :, 0]
        return 0

    jax.lax.fori_loop(0, TT, step, 0)

    @pl.when(t_blk == NT - 1)
    def _():
        mfin_ref[...] = m_s[...]


def kernel(x, M_init, W_kv, W_q, B_gates):
    wt = jnp.concatenate([W_kv, W_q], axis=0).T
    bgs = B_gates[:, :, None]
    bgl = B_gates[:, None, :]

    out, m_fin = pl.pallas_call(
        _e83_kernel,
        out_shape=(
            jax.ShapeDtypeStruct((T, B, N), jnp.float32),
            jax.ShapeDtypeStruct((K, B, N, N), jnp.float32),
        ),
        grid=(B // BB, NT),
        in_specs=[
            pl.BlockSpec((TT, BB, DIM), lambda c, t: (t, c, 0)),
            pl.BlockSpec((DIM, P), lambda c, t: (0, 0)),
            pl.BlockSpec((K, N, 1), lambda c, t: (0, 0, 0)),
            pl.BlockSpec((K, 1, N), lambda c, t: (0, 0, 0)),
            pl.BlockSpec((K, BB, N, N), lambda c, t: (0, c, 0, 0)),
        ],
        out_specs=(
            pl.BlockSpec((TT, BB, N), lambda c, t: (t, c, 0)),
            pl.BlockSpec((K, BB, N, N), lambda c, t: (0, c, 0, 0)),
        ),
        scratch_shapes=[
            pltpu.VMEM((TT, BB, P), jnp.float32),
            pltpu.VMEM((K, BB, N, N), jnp.float32),
        ],
        compiler_params=pltpu.CompilerParams(
            dimension_semantics=("parallel", "arbitrary"),
            vmem_limit_bytes=56 * 1024 * 1024,
        ),
        name="e83_tower_cell",
    )(x, wt, bgs, bgl, M_init)
    return out, m_fin


# speedup vs baseline: 2.3761x; 1.0109x over previous
import jax
import jax.numpy as jnp
from jax.experimental import pallas as pl
from jax.experimental.pallas import tpu as pltpu

T, B, DIM = 2048, 16, 1024
N, K = 64, 3
P = K * 2 * N + N
BB = 8
TT = 256
NT = T // TT


def _e83_kernel(x_ref, wt_ref, bgs_ref, bgl_ref, sinit_ref,
                out_ref, sfin_ref, proj_s, s_s):
    t_blk = pl.program_id(1)

    @pl.when(t_blk == 0)
    def _():
        s_s[...] = sinit_ref[...]

    xb = x_ref[...].reshape(TT * BB, DIM)
    proj_s[...] = jnp.dot(
        xb, wt_ref[...], preferred_element_type=jnp.float32
    ).reshape(TT, BB, P)

    def step(t, _):
        row = proj_s[t]
        qT = row[:, K * 2 * N:][:, :, None]
        s0_old = s_s[0]
        for k in range(K):
            kk = row[:, 2 * N * k: 2 * N * k + N]
            vv = row[:, 2 * N * k + N: 2 * N * (k + 1)]
            kkT = kk[:, :, None]
            nrm = jnp.sqrt(jnp.sum(kkT * kkT, axis=1, keepdims=True))
            inv = 1.0 / (nrm + 1e-6)
            knT = kkT * inv
            kn = kk * inv[:, 0, :]
            SG = s0_old if k == K - 1 else s_s[k + 1]
            Sk = s0_old if k == 0 else s_s[k]
            gk = jnp.sum(SG * knT, axis=1, keepdims=True)
            ret = jnp.sum(Sk * knT, axis=1, keepdims=True)
            cgp = jnp.sum(SG * kn[:, None, :],
                          axis=2, keepdims=True)
            rg = jax.nn.sigmoid(gk + bgl_ref[k][None])
            cg = jax.nn.sigmoid(cgp + bgs_ref[k][None])
            delta = vv[:, None, :] - ret
            s_new = cg * Sk * rg + knT * delta
            s_s[k] = s_new
            if k == 0:
                sq = jnp.sum(s_new * qT, axis=1, keepdims=True)
                o = sq * sq * jax.nn.sigmoid(sq)
                out_ref[t] = o[:, 0, :]
        return 0

    jax.lax.fori_loop(0, TT, step, 0)

    @pl.when(t_blk == NT - 1)
    def _():
        sfin_ref[...] = s_s[...]


def kernel(x, M_init, W_kv, W_q, B_gates):
    wt = jnp.concatenate([W_kv, W_q], axis=0).T
    bgs = B_gates[:, :, None]
    bgl = B_gates[:, None, :]
    s_init = M_init.transpose(0, 1, 3, 2)

    out, s_fin = pl.pallas_call(
        _e83_kernel,
        out_shape=(
            jax.ShapeDtypeStruct((T, B, N), jnp.float32),
            jax.ShapeDtypeStruct((K, B, N, N), jnp.float32),
        ),
        grid=(B // BB, NT),
        in_specs=[
            pl.BlockSpec((TT, BB, DIM), lambda c, t: (t, c, 0)),
            pl.BlockSpec((DIM, P), lambda c, t: (0, 0)),
            pl.BlockSpec((K, N, 1), lambda c, t: (0, 0, 0)),
            pl.BlockSpec((K, 1, N), lambda c, t: (0, 0, 0)),
            pl.BlockSpec((K, BB, N, N), lambda c, t: (0, c, 0, 0)),
        ],
        out_specs=(
            pl.BlockSpec((TT, BB, N), lambda c, t: (t, c, 0)),
            pl.BlockSpec((K, BB, N, N), lambda c, t: (0, c, 0, 0)),
        ),
        scratch_shapes=[
            pltpu.VMEM((TT, BB, P), jnp.float32),
            pltpu.VMEM((K, BB, N, N), jnp.float32),
        ],
        compiler_params=pltpu.CompilerParams(
            dimension_semantics=("parallel", "arbitrary"),
            vmem_limit_bytes=56 * 1024 * 1024,
        ),
        name="e83_tower_cell",
    )(x, wt, bgs, bgl, s_init)
    return out, s_fin.transpose(0, 1, 3, 2)


# speedup vs baseline: 3.0390x; 1.2790x over previous
import jax
import jax.numpy as jnp
from jax.experimental import pallas as pl
from jax.experimental.pallas import tpu as pltpu

T, B, DIM = 2048, 16, 1024
N, K = 64, 3
P = K * 2 * N + N
BB = 16
TT = 128
NT = T // TT


def _e83_kernel(x_ref, wt_ref, bgs_ref, bgl_ref, sinit_ref,
                out_ref, sfin_ref, proj_s, s_s):
    t_blk = pl.program_id(0)

    @pl.when(t_blk == 0)
    def _():
        s_s[...] = sinit_ref[...]

    xb = x_ref[...].reshape(TT * BB, DIM)
    proj_s[...] = jnp.dot(
        xb, wt_ref[...], preferred_element_type=jnp.float32
    ).reshape(TT, BB, P)

    def step(t, _):
        row = proj_s[t]
        qT = row[:, K * 2 * N:][:, :, None]
        s0_old = s_s[0]
        for k in range(K):
            kk = row[:, 2 * N * k: 2 * N * k + N]
            vv = row[:, 2 * N * k + N: 2 * N * (k + 1)]
            kkT = kk[:, :, None]
            nrm = jnp.sqrt(jnp.sum(kkT * kkT, axis=1, keepdims=True))
            inv = 1.0 / (nrm + 1e-6)
            knT = kkT * inv
            kn = kk * inv[:, 0, :]
            SG = s0_old if k == K - 1 else s_s[k + 1]
            Sk = s0_old if k == 0 else s_s[k]
            gk = jnp.sum(SG * knT, axis=1, keepdims=True)
            ret = jnp.sum(Sk * knT, axis=1, keepdims=True)
            cgp = jnp.sum(SG * kn[:, None, :],
                          axis=2, keepdims=True)
            rg = jax.nn.sigmoid(gk + bgl_ref[k][None])
            cg = jax.nn.sigmoid(cgp + bgs_ref[k][None])
            delta = vv[:, None, :] - ret
            s_new = cg * Sk * rg + knT * delta
            s_s[k] = s_new
            if k == 0:
                sq = jnp.sum(s_new * qT, axis=1, keepdims=True)
                o = sq * sq * jax.nn.sigmoid(sq)
                out_ref[t] = o[:, 0, :]
        return 0

    jax.lax.fori_loop(0, TT, step, 0)

    @pl.when(t_blk == NT - 1)
    def _():
        sfin_ref[...] = s_s[...]


def kernel(x, M_init, W_kv, W_q, B_gates):
    wt = jnp.concatenate([W_kv, W_q], axis=0).T
    bgs = B_gates[:, :, None]
    bgl = B_gates[:, None, :]
    s_init = M_init.transpose(0, 1, 3, 2)

    out, s_fin = pl.pallas_call(
        _e83_kernel,
        out_shape=(
            jax.ShapeDtypeStruct((T, B, N), jnp.float32),
            jax.ShapeDtypeStruct((K, B, N, N), jnp.float32),
        ),
        grid=(NT,),
        in_specs=[
            pl.BlockSpec((TT, BB, DIM), lambda t: (t, 0, 0)),
            pl.BlockSpec((DIM, P), lambda t: (0, 0)),
            pl.BlockSpec((K, N, 1), lambda t: (0, 0, 0)),
            pl.BlockSpec((K, 1, N), lambda t: (0, 0, 0)),
            pl.BlockSpec((K, BB, N, N), lambda t: (0, 0, 0, 0)),
        ],
        out_specs=(
            pl.BlockSpec((TT, BB, N), lambda t: (t, 0, 0)),
            pl.BlockSpec((K, BB, N, N), lambda t: (0, 0, 0, 0)),
        ),
        scratch_shapes=[
            pltpu.VMEM((TT, BB, P), jnp.float32),
            pltpu.VMEM((K, BB, N, N), jnp.float32),
        ],
        compiler_params=pltpu.CompilerParams(
            dimension_semantics=("arbitrary",),
            vmem_limit_bytes=56 * 1024 * 1024,
        ),
        name="e83_tower_cell",
    )(x, wt, bgs, bgl, s_init)
    return out, s_fin.transpose(0, 1, 3, 2)
